# baseline (speedup 1.0000x reference)
import functools

import jax
import jax.numpy as jnp
from jax import lax
from jax.experimental import pallas as pl
from jax.experimental.pallas import tpu as pltpu
from jax.experimental.pallas import tpu_sc as plsc

N = 10000
D_IN = 256
D_HID = 1024
D_OUT = 256
E = 160000

NPAD = 10240
DH = 128
C = 128
EALL = E + N
EPAD = 172032
CHUNKS = EPAD // C
DEG_CH = CHUNKS // 32
PROP_CH = CHUNKS // 16
EPADP = 196608
HALF = 48
ROWS_PT = NPAD // 16
DW = 128

_mesh = plsc.VectorSubcoreMesh(core_axis_name="c", subcore_axis_name="s")


def _make_deg_kernel(dw):
    @functools.partial(
        pl.kernel,
        out_type=jax.ShapeDtypeStruct((2, NPAD, dw), jnp.float32),
        mesh=_mesh,
        scratch_types=[
            pltpu.VMEM((PROP_CH, C), jnp.int32),
            pltpu.VMEM((C, dw), jnp.float32),
            pltpu.MemorySpace.VMEM_SHARED((NPAD, dw), jnp.float32),
        ],
    )
    def deg_kernel(dst_hbm, zeros_hbm, ones_hbm, out_hbm, idx_v, ones_v, accum):
        c = lax.axis_index("c")
        s = lax.axis_index("s")
        pltpu.sync_copy(zeros_hbm, accum.at[pl.ds(s * ROWS_PT, ROWS_PT)])
        pltpu.sync_copy(ones_hbm, ones_v)
        pltpu.sync_copy(dst_hbm.at[s], idx_v)
        plsc.subcore_barrier()

        @pl.loop(0, DEG_CH)
        def _(j):
            pltpu.sync_copy(ones_v, accum.at[idx_v.at[2 * j + c]], add=True)

        plsc.subcore_barrier()
        pltpu.sync_copy(
            accum.at[pl.ds(s * ROWS_PT, ROWS_PT)],
            out_hbm.at[c, pl.ds(s * ROWS_PT, ROWS_PT)],
        )

    return deg_kernel


_deg_kernel = _make_deg_kernel(DW)


@functools.partial(
    pl.kernel,
    out_type=(
        jax.ShapeDtypeStruct((NPAD, DH), jnp.float32),
        jax.ShapeDtypeStruct((NPAD, DH), jnp.float32),
    ),
    mesh=_mesh,
    scratch_types=[
        pltpu.VMEM((HALF, C), jnp.int32),
        pltpu.VMEM((HALF, C), jnp.int32),
        pltpu.VMEM((C, DH), jnp.float32),
        pltpu.VMEM((C, DH), jnp.float32),
        pltpu.MemorySpace.VMEM_SHARED((NPAD, DH), jnp.float32),
        pltpu.SemaphoreType.DMA,
        pltpu.SemaphoreType.DMA,
    ],
)
def _prop_kernel(src_hbm, dst_hbm, xs0_hbm, xs1_hbm, zeros_hbm,
                 out0_hbm, out1_hbm, src_v, dst_v, rows0_v, rows1_v,
                 accum, sem0, sem1):
    c = lax.axis_index("c")
    s = lax.axis_index("s")
    pltpu.sync_copy(zeros_hbm, accum.at[pl.ds(s * ROWS_PT, ROWS_PT)])
    plsc.subcore_barrier()

    def run_edges(xs_hbm):
        for h in range(2):
            pltpu.sync_copy(src_hbm.at[s, pl.ds(h * HALF, HALF)], src_v)
            pltpu.sync_copy(dst_hbm.at[s, pl.ds(h * HALF, HALF)], dst_v)

            @pl.loop(0, HALF // 2)
            def _(g):
                j = 2 * g
                d0 = pltpu.async_copy(xs_hbm.at[src_v.at[j]], rows0_v, sem0)
                d1 = pltpu.async_copy(
                    xs_hbm.at[src_v.at[j + 1]], rows1_v, sem1)
                d0.wait()
                pltpu.sync_copy(rows0_v, accum.at[dst_v.at[j]], add=True)
                d1.wait()
                pltpu.sync_copy(rows1_v, accum.at[dst_v.at[j + 1]], add=True)

    @pl.when(c == 0)
    def _():
        run_edges(xs0_hbm)

    @pl.when(c == 1)
    def _():
        run_edges(xs1_hbm)

    plsc.subcore_barrier()
    rows = pl.ds(s * ROWS_PT, ROWS_PT)

    @pl.when(c == 0)
    def _():
        pltpu.sync_copy(accum.at[rows], out0_hbm.at[rows])

    @pl.when(c == 1)
    def _():
        pltpu.sync_copy(accum.at[rows], out1_hbm.at[rows])


def _scale_body(parts_ref, x_ref, dinv_ref, xs0_ref, xs1_ref):
    deg = parts_ref[0, :, 0] + parts_ref[1, :, 0]
    dinv = jnp.where(deg > 0, lax.rsqrt(deg), 0.0)
    dinv_ref[...] = dinv
    xs = x_ref[...] * dinv[:, None]
    xs0_ref[...] = xs[:, :DH]
    xs1_ref[...] = xs[:, DH:]


def _scale(parts, x_pad):
    blk = 1024
    grid = NPAD // blk
    return pl.pallas_call(
        _scale_body,
        grid=(grid,),
        in_specs=[
            pl.BlockSpec((2, blk, DW), lambda i: (0, i, 0)),
            pl.BlockSpec((blk, D_IN), lambda i: (i, 0)),
        ],
        out_specs=[
            pl.BlockSpec((blk,), lambda i: (i,)),
            pl.BlockSpec((blk, DH), lambda i: (i, 0)),
            pl.BlockSpec((blk, DH), lambda i: (i, 0)),
        ],
        out_shape=[
            jax.ShapeDtypeStruct((NPAD,), jnp.float32),
            jax.ShapeDtypeStruct((NPAD, DH), jnp.float32),
            jax.ShapeDtypeStruct((NPAD, DH), jnp.float32),
        ],
    )(parts, x_pad)


def _mlp_body(p0_ref, p1_ref, dinv_ref, w1_ref, b1_ref, w2_ref,
              q0_ref, q1_ref):
    dinv = dinv_ref[...]
    p = jnp.concatenate([p0_ref[...], p1_ref[...]], axis=1) * dinv[:, None]
    h = jnp.maximum(
        jnp.dot(p, w1_ref[...], preferred_element_type=jnp.float32)
        + b1_ref[...][None, :],
        0.0,
    )
    q = jnp.dot(h, w2_ref[...], preferred_element_type=jnp.float32)
    q = q * dinv[:, None]
    q0_ref[...] = q[:, :DH]
    q1_ref[...] = q[:, DH:]


def _mlp(p0, p1, dinv, W1, b1, W2):
    blk = 512
    grid = NPAD // blk
    return pl.pallas_call(
        _mlp_body,
        grid=(grid,),
        in_specs=[
            pl.BlockSpec((blk, DH), lambda i: (i, 0)),
            pl.BlockSpec((blk, DH), lambda i: (i, 0)),
            pl.BlockSpec((blk,), lambda i: (i,)),
            pl.BlockSpec((D_IN, D_HID), lambda i: (0, 0)),
            pl.BlockSpec((D_HID,), lambda i: (0,)),
            pl.BlockSpec((D_HID, D_OUT), lambda i: (0, 0)),
        ],
        out_specs=[
            pl.BlockSpec((blk, DH), lambda i: (i, 0)),
            pl.BlockSpec((blk, DH), lambda i: (i, 0)),
        ],
        out_shape=[
            jax.ShapeDtypeStruct((NPAD, DH), jnp.float32),
            jax.ShapeDtypeStruct((NPAD, DH), jnp.float32),
        ],
    )(p0, p1, dinv, W1, b1, W2)


def _final_body(r0_ref, r1_ref, dinv_ref, b2_ref, out_ref):
    r = jnp.concatenate([r0_ref[...], r1_ref[...]], axis=1)
    out_ref[...] = r * dinv_ref[...][:, None] + b2_ref[...][None, :]


def _final(r0, r1, dinv, b2):
    blk = 1024
    grid = NPAD // blk
    return pl.pallas_call(
        _final_body,
        grid=(grid,),
        in_specs=[
            pl.BlockSpec((blk, DH), lambda i: (i, 0)),
            pl.BlockSpec((blk, DH), lambda i: (i, 0)),
            pl.BlockSpec((blk,), lambda i: (i,)),
            pl.BlockSpec((D_OUT,), lambda i: (0,)),
        ],
        out_specs=pl.BlockSpec((blk, D_OUT), lambda i: (i, 0)),
        out_shape=jax.ShapeDtypeStruct((NPAD, D_OUT), jnp.float32),
    )(r0, r1, dinv, b2)


@jax.jit
def _run(x, edge_index, W1, b1, W2, b2):
    i32 = jnp.int32
    loops = jnp.arange(N, dtype=i32)
    src = jnp.concatenate([
        edge_index[0].astype(i32), loops,
        jnp.zeros((EPADP - EALL,), dtype=i32),
    ])
    dst = jnp.concatenate([
        edge_index[1].astype(i32), loops,
        jnp.full((EPADP - EALL,), NPAD - 1, dtype=i32),
    ])
    src_p = src.reshape(16, 2 * HALF, C)
    dst_p = dst.reshape(16, 2 * HALF, C)
    dst_d = dst[:EPAD].reshape(16, PROP_CH, C)
    x_pad = jnp.pad(x, ((0, NPAD - N), (0, 0)))

    zeros_dw = jnp.zeros((ROWS_PT, DW), jnp.float32)
    ones_dw = jnp.ones((C, DW), jnp.float32)
    zeros_dh = jnp.zeros((ROWS_PT, DH), jnp.float32)

    parts = _deg_kernel(dst_d, zeros_dw, ones_dw)
    dinv, xs0, xs1 = _scale(parts, x_pad)
    p0, p1 = _prop_kernel(src_p, dst_p, xs0, xs1, zeros_dh)
    q0, q1 = _mlp(p0, p1, dinv, W1, b1, W2)
    r0, r1 = _prop_kernel(src_p, dst_p, q0, q1, zeros_dh)
    out = _final(r0, r1, dinv, b2)
    return out[:N]


def kernel(x, edge_index, W1, b1, W2, b2):
    return _run(x, edge_index, W1, b1, W2, b2)


# speedup vs baseline: 6.0383x; 6.0383x over previous
import functools

import jax
import jax.numpy as jnp
from jax import lax
from jax.experimental import pallas as pl
from jax.experimental.pallas import tpu as pltpu
from jax.experimental.pallas import tpu_sc as plsc

N = 10000
D_IN = 256
D_HID = 1024
D_OUT = 256
E = 160000

NPAD = 10240
DH = 128
C = 128
EALL = E + N
EPAD = 172032
CHUNKS = EPAD // C
DEG_CH = CHUNKS // 32
PROP_CH = CHUNKS // 16
EPADP = 196608
HALF = 48
ROWS_PT = NPAD // 16
DW = 128

_mesh = plsc.VectorSubcoreMesh(core_axis_name="c", subcore_axis_name="s")


def _make_deg_kernel(dw):
    @functools.partial(
        pl.kernel,
        out_type=jax.ShapeDtypeStruct((2, NPAD, dw), jnp.float32),
        mesh=_mesh,
        scratch_types=[
            pltpu.VMEM((PROP_CH, C), jnp.int32),
            pltpu.VMEM((C, dw), jnp.float32),
            pltpu.MemorySpace.VMEM_SHARED((NPAD, dw), jnp.float32),
        ],
    )
    def deg_kernel(dst_hbm, zeros_hbm, ones_hbm, out_hbm, idx_v, ones_v, accum):
        c = lax.axis_index("c")
        s = lax.axis_index("s")
        pltpu.sync_copy(zeros_hbm, accum.at[pl.ds(s * ROWS_PT, ROWS_PT)])
        pltpu.sync_copy(ones_hbm, ones_v)
        pltpu.sync_copy(dst_hbm.at[s], idx_v)
        plsc.subcore_barrier()

        @pl.loop(0, DEG_CH)
        def _(j):
            pltpu.sync_copy(ones_v, accum.at[idx_v.at[2 * j + c]], add=True)

        plsc.subcore_barrier()
        pltpu.sync_copy(
            accum.at[pl.ds(s * ROWS_PT, ROWS_PT)],
            out_hbm.at[c, pl.ds(s * ROWS_PT, ROWS_PT)],
        )

    return deg_kernel


_deg_kernel = _make_deg_kernel(DW)


@functools.partial(
    pl.kernel,
    out_type=(
        jax.ShapeDtypeStruct((NPAD, DH), jnp.float32),
        jax.ShapeDtypeStruct((NPAD, DH), jnp.float32),
    ),
    mesh=_mesh,
    scratch_types=[
        pltpu.VMEM((HALF, C), jnp.int32),
        pltpu.VMEM((HALF, C), jnp.int32),
        pltpu.VMEM((C, DH), jnp.float32),
        pltpu.VMEM((C, DH), jnp.float32),
        pltpu.MemorySpace.VMEM_SHARED((NPAD, DH), jnp.float32),
        pltpu.SemaphoreType.DMA,
        pltpu.SemaphoreType.DMA,
    ],
)
def _prop_kernel(src_hbm, dst_hbm, xs0_hbm, xs1_hbm, zeros_hbm,
                 out0_hbm, out1_hbm, src_v, dst_v, rows0_v, rows1_v,
                 accum, sem0, sem1):
    c = lax.axis_index("c")
    s = lax.axis_index("s")
    pltpu.sync_copy(zeros_hbm, accum.at[pl.ds(s * ROWS_PT, ROWS_PT)])
    plsc.subcore_barrier()

    def run_edges(xs_hbm):
        for h in range(2):
            pltpu.sync_copy(src_hbm.at[s, pl.ds(h * HALF, HALF)], src_v)
            pltpu.sync_copy(dst_hbm.at[s, pl.ds(h * HALF, HALF)], dst_v)

            @pl.loop(0, HALF // 2)
            def _(g):
                j = 2 * g
                d0 = pltpu.async_copy(xs_hbm.at[src_v.at[j]], rows0_v, sem0)
                d1 = pltpu.async_copy(
                    xs_hbm.at[src_v.at[j + 1]], rows1_v, sem1)
                d0.wait()
                pltpu.sync_copy(rows0_v, accum.at[dst_v.at[j]], add=True)
                d1.wait()
                pltpu.sync_copy(rows1_v, accum.at[dst_v.at[j + 1]], add=True)

    @pl.when(c == 0)
    def _():
        run_edges(xs0_hbm)

    @pl.when(c == 1)
    def _():
        run_edges(xs1_hbm)

    plsc.subcore_barrier()
    rows = pl.ds(s * ROWS_PT, ROWS_PT)

    @pl.when(c == 0)
    def _():
        pltpu.sync_copy(accum.at[rows], out0_hbm.at[rows])

    @pl.when(c == 1)
    def _():
        pltpu.sync_copy(accum.at[rows], out1_hbm.at[rows])


def _scale_body(parts_ref, x_ref, dinv_ref, xs0_ref, xs1_ref):
    deg = parts_ref[0, :, 0] + parts_ref[1, :, 0]
    dinv = jnp.where(deg > 0, lax.rsqrt(deg), 0.0)
    dinv_ref[...] = dinv
    xs = x_ref[...] * dinv[:, None]
    xs0_ref[...] = xs[:, :DH]
    xs1_ref[...] = xs[:, DH:]


def _scale(parts, x_pad):
    blk = 1024
    grid = NPAD // blk
    return pl.pallas_call(
        _scale_body,
        grid=(grid,),
        in_specs=[
            pl.BlockSpec((2, blk, DW), lambda i: (0, i, 0)),
            pl.BlockSpec((blk, D_IN), lambda i: (i, 0)),
        ],
        out_specs=[
            pl.BlockSpec((blk,), lambda i: (i,)),
            pl.BlockSpec((blk, DH), lambda i: (i, 0)),
            pl.BlockSpec((blk, DH), lambda i: (i, 0)),
        ],
        out_shape=[
            jax.ShapeDtypeStruct((NPAD,), jnp.float32),
            jax.ShapeDtypeStruct((NPAD, DH), jnp.float32),
            jax.ShapeDtypeStruct((NPAD, DH), jnp.float32),
        ],
    )(parts, x_pad)


def _mlp_body(p0_ref, p1_ref, dinv_ref, w1_ref, b1_ref, w2_ref,
              q0_ref, q1_ref):
    dinv = dinv_ref[...]
    p = jnp.concatenate([p0_ref[...], p1_ref[...]], axis=1) * dinv[:, None]
    h = jnp.maximum(
        jnp.dot(p, w1_ref[...], preferred_element_type=jnp.float32)
        + b1_ref[...][None, :],
        0.0,
    )
    q = jnp.dot(h, w2_ref[...], preferred_element_type=jnp.float32)
    q = q * dinv[:, None]
    q0_ref[...] = q[:, :DH]
    q1_ref[...] = q[:, DH:]


def _mlp(p0, p1, dinv, W1, b1, W2):
    blk = 512
    grid = NPAD // blk
    return pl.pallas_call(
        _mlp_body,
        grid=(grid,),
        in_specs=[
            pl.BlockSpec((blk, DH), lambda i: (i, 0)),
            pl.BlockSpec((blk, DH), lambda i: (i, 0)),
            pl.BlockSpec((blk,), lambda i: (i,)),
            pl.BlockSpec((D_IN, D_HID), lambda i: (0, 0)),
            pl.BlockSpec((D_HID,), lambda i: (0,)),
            pl.BlockSpec((D_HID, D_OUT), lambda i: (0, 0)),
        ],
        out_specs=[
            pl.BlockSpec((blk, DH), lambda i: (i, 0)),
            pl.BlockSpec((blk, DH), lambda i: (i, 0)),
        ],
        out_shape=[
            jax.ShapeDtypeStruct((NPAD, DH), jnp.float32),
            jax.ShapeDtypeStruct((NPAD, DH), jnp.float32),
        ],
    )(p0, p1, dinv, W1, b1, W2)


def _final_body(r0_ref, r1_ref, dinv_ref, b2_ref, out_ref):
    r = jnp.concatenate([r0_ref[...], r1_ref[...]], axis=1)
    out_ref[...] = r * dinv_ref[...][:, None] + b2_ref[...][None, :]


def _final(r0, r1, dinv, b2):
    blk = 1024
    grid = NPAD // blk
    return pl.pallas_call(
        _final_body,
        grid=(grid,),
        in_specs=[
            pl.BlockSpec((blk, DH), lambda i: (i, 0)),
            pl.BlockSpec((blk, DH), lambda i: (i, 0)),
            pl.BlockSpec((blk,), lambda i: (i,)),
            pl.BlockSpec((D_OUT,), lambda i: (0,)),
        ],
        out_specs=pl.BlockSpec((blk, D_OUT), lambda i: (i, 0)),
        out_shape=jax.ShapeDtypeStruct((NPAD, D_OUT), jnp.float32),
    )(r0, r1, dinv, b2)


@jax.jit
def _run(x, edge_index, W1, b1, W2, b2):
    i32 = jnp.int32
    loops = jnp.arange(N, dtype=i32)
    padc = jnp.arange(EPADP - EALL, dtype=i32)
    src = jnp.concatenate([
        edge_index[0].astype(i32), loops, padc % N,
    ])
    dst = jnp.concatenate([
        edge_index[1].astype(i32), loops, N + padc % (NPAD - N),
    ])
    src_p = src.reshape(16, 2 * HALF, C)
    dst_p = dst.reshape(16, 2 * HALF, C)
    dst_d = dst[:EPAD].reshape(16, PROP_CH, C)
    x_pad = jnp.pad(x, ((0, NPAD - N), (0, 0)))

    zeros_dw = jnp.zeros((ROWS_PT, DW), jnp.float32)
    ones_dw = jnp.ones((C, DW), jnp.float32)
    zeros_dh = jnp.zeros((ROWS_PT, DH), jnp.float32)

    parts = _deg_kernel(dst_d, zeros_dw, ones_dw)
    dinv, xs0, xs1 = _scale(parts, x_pad)
    p0, p1 = _prop_kernel(src_p, dst_p, xs0, xs1, zeros_dh)
    q0, q1 = _mlp(p0, p1, dinv, W1, b1, W2)
    r0, r1 = _prop_kernel(src_p, dst_p, q0, q1, zeros_dh)
    out = _final(r0, r1, dinv, b2)
    return out[:N]


def kernel(x, edge_index, W1, b1, W2, b2):
    return _run(x, edge_index, W1, b1, W2, b2)
